# Initial kernel scaffold; baseline (speedup 1.0000x reference)
import jax
import jax.numpy as jnp
from jax.experimental import pallas as pl


def kernel(x, gate_W, gate_b, W1, b1, ln1_g, ln1_b, W2, b2, ln2_g, ln2_b):
    raise NotImplementedError("write your pallas kernel here")


# speedup vs baseline: 1.6900x; 1.6900x over previous
import functools

import jax
import jax.numpy as jnp
from jax.experimental import pallas as pl
from jax.experimental.pallas import tpu as pltpu

N = 2048
D = 768
E = 8


def _moe_kernel(x_ref, gw_ref, gb_ref, W1_ref, b1_ref, g1_ref, be1_ref,
                W2_ref, b2_ref, g2_ref, be2_ref, out_ref, probs_ref, w_ref):
    e = pl.program_id(0)
    x = x_ref[...]

    @pl.when(e == 0)
    def _gating():
        logits = jax.lax.dot_general(
            x, gw_ref[...], (((1,), (1,)), ((), ())),
            preferred_element_type=jnp.float32) + gb_ref[...]
        m = jnp.max(logits, axis=1, keepdims=True)
        ex = jnp.exp(logits - m)
        p = ex / jnp.sum(ex, axis=1, keepdims=True)
        probs_ref[...] = p
        iota = jax.lax.broadcasted_iota(jnp.int32, (N, E), 1)
        m1 = jnp.max(p, axis=1, keepdims=True)
        idx1 = jnp.min(jnp.where(p == m1, iota, E), axis=1, keepdims=True)
        pick1 = iota == idx1
        pm = jnp.where(pick1, -1.0, p)
        m2 = jnp.max(pm, axis=1, keepdims=True)
        idx2 = jnp.min(jnp.where(pm == m2, iota, E), axis=1, keepdims=True)
        pick2 = iota == idx2
        denom = m1 + m2 + 1e-9
        w_ref[...] = jnp.where(pick1, m1 / denom,
                               jnp.where(pick2, m2 / denom, 0.0))

    iota = jax.lax.broadcasted_iota(jnp.int32, (N, E), 1)
    w_col = jnp.sum(jnp.where(iota == e, w_ref[...], 0.0), axis=1,
                    keepdims=True)

    h = jax.lax.dot_general(x, W1_ref[0], (((1,), (1,)), ((), ())),
                            preferred_element_type=jnp.float32) + b1_ref[0]
    mu = jnp.mean(h, axis=1, keepdims=True)
    var = jnp.mean((h - mu) ** 2, axis=1, keepdims=True)
    h = (h - mu) * jax.lax.rsqrt(var + 1e-5) * g1_ref[0] + be1_ref[0]
    h = jnp.maximum(h, 0.0)
    h = jax.lax.dot_general(h, W2_ref[0], (((1,), (1,)), ((), ())),
                            preferred_element_type=jnp.float32) + b2_ref[0]
    mu = jnp.mean(h, axis=1, keepdims=True)
    var = jnp.mean((h - mu) ** 2, axis=1, keepdims=True)
    h = (h - mu) * jax.lax.rsqrt(var + 1e-5) * g2_ref[0] + be2_ref[0]
    contrib = w_col * jnp.maximum(x + h, 0.0)

    @pl.when(e == 0)
    def _init():
        out_ref[...] = contrib

    @pl.when(e != 0)
    def _acc():
        out_ref[...] += contrib


@jax.jit
def kernel(x, gate_W, gate_b, W1, b1, ln1_g, ln1_b, W2, b2, ln2_g, ln2_b):
    full = lambda e: (0, 0)
    per_e3 = lambda e: (e, 0, 0)
    out, probs = pl.pallas_call(
        _moe_kernel,
        grid=(E,),
        in_specs=[
            pl.BlockSpec((N, D), full),
            pl.BlockSpec((E, D), full),
            pl.BlockSpec((1, E), full),
            pl.BlockSpec((1, D, D), per_e3),
            pl.BlockSpec((1, 1, D), per_e3),
            pl.BlockSpec((1, 1, D), per_e3),
            pl.BlockSpec((1, 1, D), per_e3),
            pl.BlockSpec((1, D, D), per_e3),
            pl.BlockSpec((1, 1, D), per_e3),
            pl.BlockSpec((1, 1, D), per_e3),
            pl.BlockSpec((1, 1, D), per_e3),
        ],
        out_specs=[
            pl.BlockSpec((N, D), full),
            pl.BlockSpec((N, E), full),
        ],
        out_shape=[
            jax.ShapeDtypeStruct((N, D), jnp.float32),
            jax.ShapeDtypeStruct((N, E), jnp.float32),
        ],
        scratch_shapes=[pltpu.VMEM((N, E), jnp.float32)],
        compiler_params=pltpu.CompilerParams(
            dimension_semantics=("arbitrary",)),
    )(x, gate_W, gate_b.reshape(1, E),
      W1, b1.reshape(E, 1, D), ln1_g.reshape(E, 1, D), ln1_b.reshape(E, 1, D),
      W2, b2.reshape(E, 1, D), ln2_g.reshape(E, 1, D), ln2_b.reshape(E, 1, D))
    return out, probs
